# Initial kernel scaffold; baseline (speedup 1.0000x reference)
import jax
import jax.numpy as jnp
from jax.experimental import pallas as pl


def kernel(x, edge_index, W_pre1, b_pre1, W_post1, b_post1, W_lin1, b_lin1, W_pre2, b_pre2, W_post2, b_post2, W_lin2, b_lin2, W_out, b_out):
    raise NotImplementedError("write your pallas kernel here")


# speedup vs baseline: 1.1386x; 1.1386x over previous
import math

import jax
import jax.numpy as jnp
from jax.experimental import pallas as pl
from jax.experimental.pallas import tpu as pltpu

D = 128
N_NODES = 10000
AVG_DEG_LOG = math.log(33.0)

_BLK = 256
_NPAD = 10240


def _combine_body(x_ref, A_ref, cnt_ref, SB_ref, S2B_ref, MN_ref, MX_ref,
                  Wp_ref, bp_ref, Wl_ref, bl_ref, o_ref):
    x = x_ref[...]
    A = A_ref[...]
    cnt = cnt_ref[...]
    SB = SB_ref[...]
    S2B = S2B_ref[...]
    MN = MN_ref[...]
    MX = MX_ref[...]

    denom = jnp.maximum(cnt, 1.0)
    inv = 1.0 / denom
    s = cnt * A + SB
    mean = s * inv
    mb = SB * inv
    varb = jnp.maximum(S2B * inv - mb * mb, 0.0)
    std = jnp.sqrt(varb + 1e-5)
    has = cnt > 0.0
    mn = jnp.where(has, A + MN, 0.0)
    mx = jnp.where(has, A + MX, 0.0)

    logd = jnp.log(denom + 1.0)
    amp = logd * (1.0 / AVG_DEG_LOG)
    att = AVG_DEG_LOG / logd

    hp = jnp.dot(x, Wp_ref[pl.ds(0, D), :], precision="highest")
    pieces = (mean, s, std, mn, mx)
    for i, p in enumerate(pieces):
        off = D + i * D
        hp += jnp.dot(p, Wp_ref[pl.ds(off, D), :], precision="highest")
    for i, p in enumerate(pieces):
        off = D + (5 + i) * D
        hp += jnp.dot(p * amp, Wp_ref[pl.ds(off, D), :], precision="highest")
    for i, p in enumerate(pieces):
        off = D + (10 + i) * D
        hp += jnp.dot(p * att, Wp_ref[pl.ds(off, D), :], precision="highest")
    hp += bp_ref[...]
    h = jnp.dot(hp, Wl_ref[...], precision="highest") + bl_ref[...]
    o_ref[...] = jnp.maximum(h, 0.0)


def _combine(x, A, cnt, SB, S2B, MN, MX, W_post, b_post, W_lin, b_lin):
    grid = (_NPAD // _BLK,)
    row_spec = pl.BlockSpec((_BLK, D), lambda i: (i, 0))
    out = pl.pallas_call(
        _combine_body,
        grid=grid,
        in_specs=[
            row_spec,
            row_spec,
            pl.BlockSpec((_BLK, 1), lambda i: (i, 0)),
            row_spec, row_spec, row_spec, row_spec,
            pl.BlockSpec((16 * D, D), lambda i: (0, 0)),
            pl.BlockSpec((1, D), lambda i: (0, 0)),
            pl.BlockSpec((D, D), lambda i: (0, 0)),
            pl.BlockSpec((1, D), lambda i: (0, 0)),
        ],
        out_specs=row_spec,
        out_shape=jax.ShapeDtypeStruct((_NPAD, D), jnp.float32),
    )(x, A, cnt, SB, S2B, MN, MX, W_post, b_post[None, :], W_lin, b_lin[None, :])
    return out


def _segment_stats(B, src, dst):
    rows = B[src]
    cnt = jax.ops.segment_sum(jnp.ones(src.shape, jnp.float32), dst,
                              num_segments=N_NODES)
    SB = jax.ops.segment_sum(rows, dst, num_segments=N_NODES)
    S2B = jax.ops.segment_sum(rows * rows, dst, num_segments=N_NODES)
    MN = jax.ops.segment_min(rows, dst, num_segments=N_NODES)
    MX = jax.ops.segment_max(rows, dst, num_segments=N_NODES)
    return cnt, SB, S2B, MN, MX


def _pad_rows(a):
    return jnp.pad(a, ((0, _NPAD - N_NODES), (0, 0)))


def _layer(x, src, dst, W_pre, b_pre, W_post, b_post, W_lin, b_lin):
    A = jnp.dot(x, W_pre[:D], precision="highest")
    B = jnp.dot(x, W_pre[D:], precision="highest") + b_pre
    cnt, SB, S2B, MN, MX = _segment_stats(B, src, dst)
    h = _combine(_pad_rows(x), _pad_rows(A), _pad_rows(cnt[:, None]),
                 _pad_rows(SB), _pad_rows(S2B), _pad_rows(MN), _pad_rows(MX),
                 W_post, b_post, W_lin, b_lin)
    return h[:N_NODES]


def kernel(x, edge_index, W_pre1, b_pre1, W_post1, b_post1, W_lin1, b_lin1,
           W_pre2, b_pre2, W_post2, b_post2, W_lin2, b_lin2, W_out, b_out):
    src = edge_index[0].astype(jnp.int32)
    dst = edge_index[1].astype(jnp.int32)
    h = _layer(x, src, dst, W_pre1, b_pre1, W_post1, b_post1, W_lin1, b_lin1)
    h = _layer(h, src, dst, W_pre2, b_pre2, W_post2, b_post2, W_lin2, b_lin2)
    out = jnp.dot(h, W_out, precision="highest") + b_out
    return jnp.squeeze(out, axis=-1)


# speedup vs baseline: 2.9700x; 2.6083x over previous
import functools
import math

import jax
import jax.numpy as jnp
from jax import lax
from jax.experimental import pallas as pl
from jax.experimental.pallas import tpu as pltpu
from jax.experimental.pallas import tpu_sc as plsc

D = 128
HALF = 64
N_NODES = 10000
N_EDGES = 320000
AVG_DEG_LOG = math.log(33.0)

_BLK = 256
_NPAD = 10240

_NT = 320
_NPASS = 2
_CH = 4000
_NCH = N_EDGES // _CH
_CHV = _CH // 16
_G = 128
_MROWS = 34
_ACC = _NT * HALF
_ACC2 = (_NT + 1) * HALF

_FMAX = 3.4e38


def _sc_body(dst_hbm, src_hbm, B_hbm,
             cnt_out, S_out, S2_out, MN_out, MX_out,
             dstc, srcc, idxb, dlb, rowb,
             Sa, S2a, MNa, MXa, cnta, sem):
    c = lax.axis_index("c")
    s = lax.axis_index("s")
    ones = jnp.ones((16,), jnp.float32)
    zf = jnp.zeros((16,), jnp.float32)

    def flush(nb, mcnt):
        def batch_body(b, _):
            pltpu.async_copy(B_hbm.at[idxb.at[b]], rowb, sem).wait()
            ng = (jnp.minimum(mcnt - b * _G, _G) + 15) >> 4

            def group_body(g, _):
                dlv = dlb[pl.ds(b * _G + g * 16, 16)]
                for j in range(16):
                    base = dlv[j] * HALF
                    for k in range(4):
                        off = pl.ds(base + k * 16, 16)
                        rv = rowb[g * 16 + j, pl.ds(k * 16, 16)]
                        plsc.addupdate(Sa.at[off], rv)
                        plsc.addupdate(S2a.at[off], rv * rv)
                        MNa[off] = jnp.minimum(MNa[off], rv)
                        MXa[off] = jnp.maximum(MXa[off], rv)
                return 0

            lax.fori_loop(0, ng, group_body, 0)
            return 0

        lax.fori_loop(0, nb, batch_body, 0)

    def pass_body(p, _):
        r = p * 16 + s
        lo = r * _NT

        def zero_body(i, _):
            off = pl.ds(i * 16, 16)
            Sa[off] = zf
            S2a[off] = zf
            MNa[off] = zf + _FMAX
            MXa[off] = zf - _FMAX
            return 0

        lax.fori_loop(0, _ACC2 // 16, zero_body, 0)

        def zero_cnt(i, _):
            cnta[pl.ds(i * 16, 16)] = zf
            return 0

        lax.fori_loop(0, _NT // 16, zero_cnt, 0)

        def chunk_body(ci, mcnt):
            pltpu.sync_copy(dst_hbm.at[pl.ds(ci * _CH, _CH)], dstc)
            pltpu.sync_copy(src_hbm.at[pl.ds(ci * _CH, _CH)], srcc)

            def vreg_body(v, mcnt):
                dstv = dstc[pl.ds(v * 16, 16)]
                srcv = srcc[pl.ds(v * 16, 16)]
                match = (dstv >= lo) & (dstv < lo + _NT)
                dloc = dstv - lo
                plsc.addupdate_scatter(cnta, [dloc], ones,
                                       mask=match)
                mi = jnp.where(match, 1, 0)
                ranks = plsc.cumsum(mi)
                pos = ranks + (mcnt - 1)
                idxv = srcv + c * N_NODES
                plsc.store_scatter(idxb,
                                   [pos >> 7, pos & 127],
                                   idxv, mask=match)
                plsc.store_scatter(dlb, [pos], dloc, mask=match)
                return mcnt + jnp.sum(mi)

            mcnt = lax.fori_loop(0, _CHV, vreg_body, mcnt)
            nb = mcnt >> 7
            flush(nb, mcnt)
            for t in range(8):
                tsl = pl.ds(t * 16, 16)
                idxb[0, tsl] = idxb[nb, tsl]
                dlb[tsl] = dlb[pl.ds(nb * _G + t * 16, 16)]
            return mcnt & 127

        mcnt = lax.fori_loop(0, _NCH, chunk_body, 0)
        lanes = lax.iota(jnp.int32, 16)
        padto = ((mcnt + 15) >> 4) << 4
        ppos = mcnt + lanes
        pmask = ppos < padto
        plsc.store_scatter(idxb, [ppos >> 7, ppos & 127],
                           jnp.zeros((16,), jnp.int32), mask=pmask)
        plsc.store_scatter(dlb, [ppos],
                           jnp.full((16,), _NT, jnp.int32), mask=pmask)
        flush((mcnt + 127) >> 7, mcnt)

        asl = pl.ds(0, _ACC)
        pltpu.sync_copy(cnta, cnt_out.at[c, r])
        pltpu.sync_copy(Sa.at[asl], S_out.at[c, r])
        pltpu.sync_copy(S2a.at[asl], S2_out.at[c, r])
        pltpu.sync_copy(MNa.at[asl], MN_out.at[c, r])
        pltpu.sync_copy(MXa.at[asl], MX_out.at[c, r])
        return 0

    lax.fori_loop(0, _NPASS, pass_body, 0)


_sc_stats_call = functools.partial(
    pl.kernel,
    mesh=plsc.VectorSubcoreMesh(core_axis_name="c", subcore_axis_name="s"),
    compiler_params=pltpu.CompilerParams(needs_layout_passes=False,
                                         use_tc_tiling_on_sc=False),
    out_type=[
        jax.ShapeDtypeStruct((2, 32, _NT), jnp.float32),
        jax.ShapeDtypeStruct((2, 32, _ACC), jnp.float32),
        jax.ShapeDtypeStruct((2, 32, _ACC), jnp.float32),
        jax.ShapeDtypeStruct((2, 32, _ACC), jnp.float32),
        jax.ShapeDtypeStruct((2, 32, _ACC), jnp.float32),
    ],
    scratch_types=[
        pltpu.VMEM((_CH,), jnp.int32),
        pltpu.VMEM((_CH,), jnp.int32),
        pltpu.VMEM((_MROWS, _G), jnp.int32),
        pltpu.VMEM((_MROWS * _G,), jnp.int32),
        pltpu.VMEM((_G, HALF), jnp.float32),
        pltpu.VMEM((_ACC2,), jnp.float32),
        pltpu.VMEM((_ACC2,), jnp.float32),
        pltpu.VMEM((_ACC2,), jnp.float32),
        pltpu.VMEM((_ACC2,), jnp.float32),
        pltpu.VMEM((_NT,), jnp.float32),
        pltpu.SemaphoreType.DMA,
    ],
)(_sc_body)


def _sc_stats(dst, src, B):
    Bstack = jnp.concatenate([B[:, :HALF], B[:, HALF:]], axis=0)
    cnt2, S, S2, MN, MX = _sc_stats_call(dst, src, Bstack)
    cnt = cnt2.reshape(2, _NPAD)[0]

    def halves(a):
        a = a.reshape(2, _NPAD, HALF)
        return jnp.concatenate([a[0], a[1]], axis=-1)

    return cnt, halves(S), halves(S2), halves(MN), halves(MX)


def _combine_body(x_ref, A_ref, cnt_ref, SB_ref, S2B_ref, MN_ref, MX_ref,
                  Wp_ref, bp_ref, Wl_ref, bl_ref, o_ref):
    x = x_ref[...]
    A = A_ref[...]
    cnt = cnt_ref[...]
    SB = SB_ref[...]
    S2B = S2B_ref[...]
    MN = MN_ref[...]
    MX = MX_ref[...]

    denom = jnp.maximum(cnt, 1.0)
    inv = 1.0 / denom
    s = cnt * A + SB
    mean = s * inv
    mb = SB * inv
    varb = jnp.maximum(S2B * inv - mb * mb, 0.0)
    std = jnp.sqrt(varb + 1e-5)
    has = cnt > 0.0
    mn = jnp.where(has, A + MN, 0.0)
    mx = jnp.where(has, A + MX, 0.0)

    logd = jnp.log(denom + 1.0)
    amp = logd * (1.0 / AVG_DEG_LOG)
    att = AVG_DEG_LOG / logd

    hp = jnp.dot(x, Wp_ref[pl.ds(0, D), :], precision="default")
    pieces = (mean, s, std, mn, mx)
    for i, p in enumerate(pieces):
        off = D + i * D
        hp += jnp.dot(p, Wp_ref[pl.ds(off, D), :], precision="default")
    for i, p in enumerate(pieces):
        off = D + (5 + i) * D
        hp += jnp.dot(p * amp, Wp_ref[pl.ds(off, D), :], precision="default")
    for i, p in enumerate(pieces):
        off = D + (10 + i) * D
        hp += jnp.dot(p * att, Wp_ref[pl.ds(off, D), :], precision="default")
    hp += bp_ref[...]
    h = jnp.dot(hp, Wl_ref[...], precision="default") + bl_ref[...]
    o_ref[...] = jnp.maximum(h, 0.0)


def _combine(x, A, cnt, SB, S2B, MN, MX, W_post, b_post, W_lin, b_lin):
    grid = (_NPAD // _BLK,)
    row_spec = pl.BlockSpec((_BLK, D), lambda i: (i, 0))
    out = pl.pallas_call(
        _combine_body,
        grid=grid,
        in_specs=[
            row_spec,
            row_spec,
            pl.BlockSpec((_BLK, 1), lambda i: (i, 0)),
            row_spec, row_spec, row_spec, row_spec,
            pl.BlockSpec((16 * D, D), lambda i: (0, 0)),
            pl.BlockSpec((1, D), lambda i: (0, 0)),
            pl.BlockSpec((D, D), lambda i: (0, 0)),
            pl.BlockSpec((1, D), lambda i: (0, 0)),
        ],
        out_specs=row_spec,
        out_shape=jax.ShapeDtypeStruct((_NPAD, D), jnp.float32),
    )(x, A, cnt, SB, S2B, MN, MX, W_post, b_post[None, :], W_lin, b_lin[None, :])
    return out


def _pad_rows(a):
    return jnp.pad(a, ((0, _NPAD - N_NODES), (0, 0)))


def _layer(x, src, dst, W_pre, b_pre, W_post, b_post, W_lin, b_lin):
    A = jnp.dot(x, W_pre[:D], precision="default")
    B = jnp.dot(x, W_pre[D:], precision="default") + b_pre
    cnt, SB, S2B, MN, MX = _sc_stats(dst, src, B)
    h = _combine(_pad_rows(x), _pad_rows(A), cnt[:, None],
                 SB, S2B, MN, MX,
                 W_post, b_post, W_lin, b_lin)
    return h[:N_NODES]


def kernel(x, edge_index, W_pre1, b_pre1, W_post1, b_post1, W_lin1, b_lin1,
           W_pre2, b_pre2, W_post2, b_post2, W_lin2, b_lin2, W_out, b_out):
    src = edge_index[0].astype(jnp.int32)
    dst = edge_index[1].astype(jnp.int32)
    h = _layer(x, src, dst, W_pre1, b_pre1, W_post1, b_post1, W_lin1, b_lin1)
    h = _layer(h, src, dst, W_pre2, b_pre2, W_post2, b_post2, W_lin2, b_lin2)
    out = jnp.dot(h, W_out, precision="default") + b_out
    return jnp.squeeze(out, axis=-1)


# speedup vs baseline: 3.7179x; 1.2518x over previous
import functools
import math

import jax
import jax.numpy as jnp
from jax import lax
from jax.experimental import pallas as pl
from jax.experimental.pallas import tpu as pltpu
from jax.experimental.pallas import tpu_sc as plsc

D = 128
HALF = 64
N_NODES = 10000
N_EDGES = 320000
AVG_DEG_LOG = math.log(33.0)

_BLK = 256
_NPAD = 10240

_NT = 320
_NPASS = 2
_CH = 3200
_NCH = N_EDGES // _CH
_CHV = _CH // 16
_G = 128
_MROWS = 32
_ACC = _NT * HALF
_ACC2 = (_NT + 1) * HALF

_FMAX = 3.4e38


def _sc_body(dst_hbm, src_hbm, B_hbm,
             cnt_out, S_out, S2_out, MN_out, MX_out,
             dstc, srcc, idxb, dlb, rowb,
             Sa, S2a, MNa, MXa, cnta,
             gsem0, gsem1, csem):
    c = lax.axis_index("c")
    s = lax.axis_index("s")
    ones = jnp.ones((16,), jnp.float32)
    zf = jnp.zeros((16,), jnp.float32)

    def chunk_descs(ci):
        slot = ci & 1
        hsl = pl.ds(ci * _CH, _CH)
        return (pltpu.make_async_copy(dst_hbm.at[hsl], dstc.at[slot], csem),
                pltpu.make_async_copy(src_hbm.at[hsl], srcc.at[slot], csem))

    def chunk_start(ci):
        for d in chunk_descs(ci):
            d.start()

    def chunk_wait(ci):
        for d in chunk_descs(ci):
            d.wait()

    def gather_start(b):
        row = b & (_MROWS - 1)

        def go(slot_sem, slot):
            pltpu.make_async_copy(
                B_hbm.at[idxb.at[row]], rowb.at[slot], slot_sem).start()

        pl.when((b & 1) == 0)(lambda: go(gsem0, 0))
        pl.when((b & 1) == 1)(lambda: go(gsem1, 1))

    def accumulate(b, ng):
        row = b & (_MROWS - 1)
        slot = b & 1

        def wait(slot_sem, sl):
            pltpu.make_async_copy(
                B_hbm.at[idxb.at[row]], rowb.at[sl], slot_sem).wait()

        pl.when((b & 1) == 0)(lambda: wait(gsem0, 0))
        pl.when((b & 1) == 1)(lambda: wait(gsem1, 1))

        def group_body(g, _):
            dlv = dlb[pl.ds(row * _G + g * 16, 16)]
            for j in range(16):
                base = dlv[j] * HALF
                for k in range(4):
                    off = pl.ds(base + k * 16, 16)
                    rv = rowb[slot, g * 16 + j, pl.ds(k * 16, 16)]
                    plsc.addupdate(Sa.at[off], rv)
                    plsc.addupdate(S2a.at[off], rv * rv)
                    MNa[off] = jnp.minimum(MNa[off], rv)
                    MXa[off] = jnp.maximum(MXa[off], rv)
            return 0

        lax.fori_loop(0, ng, group_body, 0)

    def flush(gdone, target, mcnt):
        def batch_body(b, _):
            gather_start(b)
            pl.when(b > 0)(lambda: accumulate(b - 1, _G // 16))
            return 0

        lax.fori_loop(gdone, target, batch_body, 0)

    def pass_body(p, _):
        r = p * 16 + s
        lo = r * _NT

        def zero_body(i, _):
            off = pl.ds(i * 16, 16)
            Sa[off] = zf
            S2a[off] = zf
            MNa[off] = zf + _FMAX
            MXa[off] = zf - _FMAX
            return 0

        lax.fori_loop(0, _ACC2 // 16, zero_body, 0)

        def zero_cnt(i, _):
            cnta[pl.ds(i * 16, 16)] = zf
            return 0

        lax.fori_loop(0, _NT // 16, zero_cnt, 0)

        chunk_start(0)

        def chunk_body(ci, carry):
            mcnt, gdone = carry
            chunk_wait(ci)
            pl.when(ci + 1 < _NCH)(lambda: chunk_start(ci + 1))
            slot = ci & 1

            def vreg_body(v, mcnt):
                dstv = dstc[slot, pl.ds(v * 16, 16)]
                srcv = srcc[slot, pl.ds(v * 16, 16)]
                match = (dstv >= lo) & (dstv < lo + _NT)
                dloc = dstv - lo
                plsc.addupdate_scatter(cnta, [dloc], ones, mask=match)
                mi = jnp.where(match, 1, 0)
                ranks = plsc.cumsum(mi)
                pos = ranks + (mcnt - 1)
                idxv = srcv + c * N_NODES
                plsc.store_scatter(idxb,
                                   [(pos >> 7) & (_MROWS - 1), pos & 127],
                                   idxv, mask=match)
                plsc.store_scatter(dlb, [pos & (_MROWS * _G - 1)],
                                   dloc, mask=match)
                return mcnt + ranks[15]

            mcnt = lax.fori_loop(0, _CHV, vreg_body, mcnt)
            target = mcnt >> 7
            flush(gdone, target, mcnt)
            return mcnt, jnp.maximum(gdone, target)

        mcnt, gdone = lax.fori_loop(0, _NCH, chunk_body, (0, 0))
        lanes = lax.iota(jnp.int32, 16)
        padto = ((mcnt + 15) >> 4) << 4
        ppos = mcnt + lanes
        pmask = ppos < padto
        plsc.store_scatter(idxb, [(ppos >> 7) & (_MROWS - 1), ppos & 127],
                           jnp.zeros((16,), jnp.int32), mask=pmask)
        plsc.store_scatter(dlb, [ppos & (_MROWS * _G - 1)],
                           jnp.full((16,), _NT, jnp.int32), mask=pmask)
        targf = (mcnt + 127) >> 7
        flush(gdone, targf, mcnt)
        pl.when(targf > 0)(
            lambda: accumulate(targf - 1,
                               (jnp.minimum(mcnt - (targf - 1) * _G, _G)
                                + 15) >> 4))

        asl = pl.ds(0, _ACC)
        pltpu.sync_copy(cnta, cnt_out.at[c, r])
        pltpu.sync_copy(Sa.at[asl], S_out.at[c, r])
        pltpu.sync_copy(S2a.at[asl], S2_out.at[c, r])
        pltpu.sync_copy(MNa.at[asl], MN_out.at[c, r])
        pltpu.sync_copy(MXa.at[asl], MX_out.at[c, r])
        return 0

    lax.fori_loop(0, _NPASS, pass_body, 0)


_sc_stats_call = functools.partial(
    pl.kernel,
    mesh=plsc.VectorSubcoreMesh(core_axis_name="c", subcore_axis_name="s"),
    compiler_params=pltpu.CompilerParams(needs_layout_passes=False,
                                         use_tc_tiling_on_sc=False),
    out_type=[
        jax.ShapeDtypeStruct((2, 32, _NT), jnp.float32),
        jax.ShapeDtypeStruct((2, 32, _ACC), jnp.float32),
        jax.ShapeDtypeStruct((2, 32, _ACC), jnp.float32),
        jax.ShapeDtypeStruct((2, 32, _ACC), jnp.float32),
        jax.ShapeDtypeStruct((2, 32, _ACC), jnp.float32),
    ],
    scratch_types=[
        pltpu.VMEM((2, _CH), jnp.int32),
        pltpu.VMEM((2, _CH), jnp.int32),
        pltpu.VMEM((_MROWS, _G), jnp.int32),
        pltpu.VMEM((_MROWS * _G,), jnp.int32),
        pltpu.VMEM((2, _G, HALF), jnp.float32),
        pltpu.VMEM((_ACC2,), jnp.float32),
        pltpu.VMEM((_ACC2,), jnp.float32),
        pltpu.VMEM((_ACC2,), jnp.float32),
        pltpu.VMEM((_ACC2,), jnp.float32),
        pltpu.VMEM((_NT,), jnp.float32),
        pltpu.SemaphoreType.DMA,
        pltpu.SemaphoreType.DMA,
        pltpu.SemaphoreType.DMA,
    ],
)(_sc_body)


def _sc_stats(dst, src, B):
    Bstack = jnp.concatenate([B[:, :HALF], B[:, HALF:]], axis=0)
    cnt2, S, S2, MN, MX = _sc_stats_call(dst, src, Bstack)
    cnt = cnt2.reshape(2, _NPAD)[0]

    def halves(a):
        a = a.reshape(2, _NPAD, HALF)
        return jnp.concatenate([a[0], a[1]], axis=-1)

    return cnt, halves(S), halves(S2), halves(MN), halves(MX)


def _combine_body(x_ref, A_ref, cnt_ref, SB_ref, S2B_ref, MN_ref, MX_ref,
                  Wp_ref, bp_ref, Wl_ref, bl_ref, o_ref):
    x = x_ref[...]
    A = A_ref[...]
    cnt = cnt_ref[...]
    SB = SB_ref[...]
    S2B = S2B_ref[...]
    MN = MN_ref[...]
    MX = MX_ref[...]

    denom = jnp.maximum(cnt, 1.0)
    inv = 1.0 / denom
    s = cnt * A + SB
    mean = s * inv
    mb = SB * inv
    varb = jnp.maximum(S2B * inv - mb * mb, 0.0)
    std = jnp.sqrt(varb + 1e-5)
    has = cnt > 0.0
    mn = jnp.where(has, A + MN, 0.0)
    mx = jnp.where(has, A + MX, 0.0)

    logd = jnp.log(denom + 1.0)
    amp = logd * (1.0 / AVG_DEG_LOG)
    att = AVG_DEG_LOG / logd

    hp = jnp.dot(x, Wp_ref[pl.ds(0, D), :], precision="default")
    pieces = (mean, s, std, mn, mx)
    for i, p in enumerate(pieces):
        off = D + i * D
        hp += jnp.dot(p, Wp_ref[pl.ds(off, D), :], precision="default")
    for i, p in enumerate(pieces):
        off = D + (5 + i) * D
        hp += jnp.dot(p * amp, Wp_ref[pl.ds(off, D), :], precision="default")
    for i, p in enumerate(pieces):
        off = D + (10 + i) * D
        hp += jnp.dot(p * att, Wp_ref[pl.ds(off, D), :], precision="default")
    hp += bp_ref[...]
    h = jnp.dot(hp, Wl_ref[...], precision="default") + bl_ref[...]
    o_ref[...] = jnp.maximum(h, 0.0)


def _combine(x, A, cnt, SB, S2B, MN, MX, W_post, b_post, W_lin, b_lin):
    grid = (_NPAD // _BLK,)
    row_spec = pl.BlockSpec((_BLK, D), lambda i: (i, 0))
    out = pl.pallas_call(
        _combine_body,
        grid=grid,
        in_specs=[
            row_spec,
            row_spec,
            pl.BlockSpec((_BLK, 1), lambda i: (i, 0)),
            row_spec, row_spec, row_spec, row_spec,
            pl.BlockSpec((16 * D, D), lambda i: (0, 0)),
            pl.BlockSpec((1, D), lambda i: (0, 0)),
            pl.BlockSpec((D, D), lambda i: (0, 0)),
            pl.BlockSpec((1, D), lambda i: (0, 0)),
        ],
        out_specs=row_spec,
        out_shape=jax.ShapeDtypeStruct((_NPAD, D), jnp.float32),
    )(x, A, cnt, SB, S2B, MN, MX, W_post, b_post[None, :], W_lin, b_lin[None, :])
    return out


def _pad_rows(a):
    return jnp.pad(a, ((0, _NPAD - N_NODES), (0, 0)))


def _layer(x, src, dst, W_pre, b_pre, W_post, b_post, W_lin, b_lin):
    A = jnp.dot(x, W_pre[:D], precision="default")
    B = jnp.dot(x, W_pre[D:], precision="default") + b_pre
    cnt, SB, S2B, MN, MX = _sc_stats(dst, src, B)
    h = _combine(_pad_rows(x), _pad_rows(A), cnt[:, None],
                 SB, S2B, MN, MX,
                 W_post, b_post, W_lin, b_lin)
    return h[:N_NODES]


def kernel(x, edge_index, W_pre1, b_pre1, W_post1, b_post1, W_lin1, b_lin1,
           W_pre2, b_pre2, W_post2, b_post2, W_lin2, b_lin2, W_out, b_out):
    src = edge_index[0].astype(jnp.int32)
    dst = edge_index[1].astype(jnp.int32)
    h = _layer(x, src, dst, W_pre1, b_pre1, W_post1, b_post1, W_lin1, b_lin1)
    h = _layer(h, src, dst, W_pre2, b_pre2, W_post2, b_post2, W_lin2, b_lin2)
    out = jnp.dot(h, W_out, precision="default") + b_out
    return jnp.squeeze(out, axis=-1)


# speedup vs baseline: 5.3533x; 1.4399x over previous
import functools
import math

import jax
import jax.numpy as jnp
from jax import lax
from jax.experimental import pallas as pl
from jax.experimental.pallas import tpu as pltpu
from jax.experimental.pallas import tpu_sc as plsc

D = 128
HALF = 64
N_NODES = 10000
N_EDGES = 320000
AVG_DEG_LOG = math.log(33.0)

_BLK = 256
_NPAD = 10240

_NT = 320
_CH = 3200
_NCH = N_EDGES // _CH
_CHV = _CH // 16
_G = 128
_MROWS = 32
_ACC = _NT * HALF
_ACC2 = (_NT + 1) * HALF

_FMAX = 3.4e38


def _sc_build_body(dst_hbm, src_hbm,
                   cnt_out, mc_out, msrc_out, mdl_out,
                   dstc, srcc, idxb, dlb, mcstg, cnta,
                   csem, wsem):
    c = lax.axis_index("c")
    s = lax.axis_index("s")
    r = c * 16 + s
    lo = r * _NT
    ones = jnp.ones((16,), jnp.float32)
    zf = jnp.zeros((16,), jnp.float32)

    def chunk_descs(ci):
        slot = ci & 1
        hsl = pl.ds(ci * _CH, _CH)
        return (pltpu.make_async_copy(dst_hbm.at[hsl], dstc.at[slot], csem),
                pltpu.make_async_copy(src_hbm.at[hsl], srcc.at[slot], csem))

    def chunk_start(ci):
        for dsc in chunk_descs(ci):
            dsc.start()

    def chunk_wait(ci):
        for dsc in chunk_descs(ci):
            dsc.wait()

    def write_descs(b):
        row = b & (_MROWS - 1)
        bsl = pl.ds(b * _G, _G)
        return (pltpu.make_async_copy(idxb.at[row], msrc_out.at[r, bsl],
                                      wsem),
                pltpu.make_async_copy(dlb.at[pl.ds(row * _G, _G)],
                                      mdl_out.at[r, bsl], wsem))

    def write_start(b):
        for dsc in write_descs(b):
            dsc.start()

    def write_wait(b):
        for dsc in write_descs(b):
            dsc.wait()

    def zero_cnt(i, _):
        cnta[pl.ds(i * 16, 16)] = zf
        return 0

    lax.fori_loop(0, _NT // 16, zero_cnt, 0)

    chunk_start(0)

    def wb_body(b, _):
        write_start(b)
        pl.when(b > 0)(lambda: write_wait(b - 1))
        return 0

    def chunk_body(ci, carry):
        mcnt, wdone = carry
        chunk_wait(ci)
        pl.when(ci + 1 < _NCH)(lambda: chunk_start(ci + 1))
        slot = ci & 1

        def vreg_body(v, mcnt):
            dstv = dstc[slot, pl.ds(v * 16, 16)]
            srcv = srcc[slot, pl.ds(v * 16, 16)]
            match = (dstv >= lo) & (dstv < lo + _NT)
            dloc = dstv - lo
            plsc.addupdate_scatter(cnta, [dloc], ones, mask=match)
            mi = jnp.where(match, 1, 0)
            ranks = plsc.cumsum(mi)
            pos = ranks + (mcnt - 1)
            plsc.store_scatter(idxb,
                               [(pos >> 7) & (_MROWS - 1), pos & 127],
                               srcv, mask=match)
            plsc.store_scatter(dlb, [pos & (_MROWS * _G - 1)],
                               dloc, mask=match)
            return mcnt + ranks[15]

        mcnt = lax.fori_loop(0, _CHV, vreg_body, mcnt)
        target = mcnt >> 7
        lax.fori_loop(wdone, target, wb_body, 0)
        return mcnt, jnp.maximum(wdone, target)

    mcnt, wdone = lax.fori_loop(0, _NCH, chunk_body, (0, 0))

    lanes = lax.iota(jnp.int32, 16)
    padto = ((mcnt + 15) >> 4) << 4
    ppos = mcnt + lanes
    pmask = ppos < padto
    plsc.store_scatter(idxb, [(ppos >> 7) & (_MROWS - 1), ppos & 127],
                       jnp.zeros((16,), jnp.int32), mask=pmask)
    plsc.store_scatter(dlb, [ppos & (_MROWS * _G - 1)],
                       jnp.full((16,), _NT, jnp.int32), mask=pmask)
    targf = (padto + 127) >> 7
    lax.fori_loop(wdone, targf, wb_body, 0)
    pl.when(targf > 0)(lambda: write_wait(targf - 1))

    mcstg[...] = jnp.zeros((16,), jnp.int32) + padto
    pltpu.sync_copy(mcstg, mc_out.at[r])
    pltpu.sync_copy(cnta, cnt_out.at[r])


_sc_build_call = functools.partial(
    pl.kernel,
    mesh=plsc.VectorSubcoreMesh(core_axis_name="c", subcore_axis_name="s"),
    compiler_params=pltpu.CompilerParams(needs_layout_passes=False,
                                         use_tc_tiling_on_sc=False),
    out_type=[
        jax.ShapeDtypeStruct((32, _NT), jnp.float32),
        jax.ShapeDtypeStruct((32, 16), jnp.int32),
        jax.ShapeDtypeStruct((32, N_EDGES), jnp.int32),
        jax.ShapeDtypeStruct((32, N_EDGES), jnp.int32),
    ],
    scratch_types=[
        pltpu.VMEM((2, _CH), jnp.int32),
        pltpu.VMEM((2, _CH), jnp.int32),
        pltpu.VMEM((_MROWS, _G), jnp.int32),
        pltpu.VMEM((_MROWS * _G,), jnp.int32),
        pltpu.VMEM((16,), jnp.int32),
        pltpu.VMEM((_NT,), jnp.float32),
        pltpu.SemaphoreType.DMA,
        pltpu.SemaphoreType.DMA,
    ],
)(_sc_build_body)


def _sc_replay_body(mc_hbm, msrc_hbm, mdl_hbm, B_hbm,
                    S_out, S2_out, MN_out, MX_out,
                    lsrc, ldl, gidx, rowb, mcb,
                    Sa, S2a, MNa, MXa,
                    lsem, gsem0, gsem1):
    c = lax.axis_index("c")
    s = lax.axis_index("s")
    zf = jnp.zeros((16,), jnp.float32)

    def pass_body(p, _):
        r = p * 16 + s

        def zero_body(i, _):
            off = pl.ds(i * 16, 16)
            Sa[off] = zf
            S2a[off] = zf
            MNa[off] = zf + _FMAX
            MXa[off] = zf - _FMAX
            return 0

        lax.fori_loop(0, _ACC2 // 16, zero_body, 0)

        pltpu.sync_copy(mc_hbm.at[r], mcb)
        padcnt = mcb[...][0]
        nb = (padcnt + 127) >> 7

        def list_descs(b):
            slot = b & 1
            bsl = pl.ds(b * _G, _G)
            return (pltpu.make_async_copy(msrc_hbm.at[r, bsl],
                                          lsrc.at[slot], lsem),
                    pltpu.make_async_copy(mdl_hbm.at[r, bsl],
                                          ldl.at[slot], lsem))

        def lstart(b):
            for dsc in list_descs(b):
                dsc.start()

        def lwait(b):
            for dsc in list_descs(b):
                dsc.wait()

        def prep_and_gather(b):
            slot = b & 1
            lwait(b)
            for t in range(8):
                tsl = pl.ds(t * 16, 16)
                gidx[slot, tsl] = lsrc[slot, tsl] + c * N_NODES

            def go(slot_sem, sl):
                pltpu.make_async_copy(
                    B_hbm.at[gidx.at[sl]], rowb.at[sl], slot_sem).start()

            pl.when((b & 1) == 0)(lambda: go(gsem0, 0))
            pl.when((b & 1) == 1)(lambda: go(gsem1, 1))

        def accumulate(b):
            slot = b & 1

            def wait(slot_sem, sl):
                pltpu.make_async_copy(
                    B_hbm.at[gidx.at[sl]], rowb.at[sl], slot_sem).wait()

            pl.when((b & 1) == 0)(lambda: wait(gsem0, 0))
            pl.when((b & 1) == 1)(lambda: wait(gsem1, 1))
            ng = jnp.minimum(padcnt - b * _G, _G) >> 4

            def group_body(g, _):
                dlv = ldl[slot, pl.ds(g * 16, 16)]
                for j in range(16):
                    base = dlv[j] * HALF
                    for k in range(4):
                        off = pl.ds(base + k * 16, 16)
                        rv = rowb[slot, g * 16 + j, pl.ds(k * 16, 16)]
                        plsc.addupdate(Sa.at[off], rv)
                        plsc.addupdate(S2a.at[off], rv * rv)
                        MNa[off] = jnp.minimum(MNa[off], rv)
                        MXa[off] = jnp.maximum(MXa[off], rv)
                return 0

            lax.fori_loop(0, ng, group_body, 0)

        def step(j, _):
            pl.when(j >= 2)(lambda: accumulate(j - 2))
            pl.when(j < nb)(lambda: lstart(j))
            pl.when((j >= 1) & (j <= nb))(lambda: prep_and_gather(j - 1))
            return 0

        lax.fori_loop(0, nb + 2, step, 0)

        asl = pl.ds(0, _ACC)
        pltpu.sync_copy(Sa.at[asl], S_out.at[c, r])
        pltpu.sync_copy(S2a.at[asl], S2_out.at[c, r])
        pltpu.sync_copy(MNa.at[asl], MN_out.at[c, r])
        pltpu.sync_copy(MXa.at[asl], MX_out.at[c, r])
        return 0

    lax.fori_loop(0, 2, pass_body, 0)


_sc_replay_call = functools.partial(
    pl.kernel,
    mesh=plsc.VectorSubcoreMesh(core_axis_name="c", subcore_axis_name="s"),
    compiler_params=pltpu.CompilerParams(needs_layout_passes=False,
                                         use_tc_tiling_on_sc=False),
    out_type=[
        jax.ShapeDtypeStruct((2, 32, _ACC), jnp.float32),
        jax.ShapeDtypeStruct((2, 32, _ACC), jnp.float32),
        jax.ShapeDtypeStruct((2, 32, _ACC), jnp.float32),
        jax.ShapeDtypeStruct((2, 32, _ACC), jnp.float32),
    ],
    scratch_types=[
        pltpu.VMEM((2, _G), jnp.int32),
        pltpu.VMEM((2, _G), jnp.int32),
        pltpu.VMEM((2, _G), jnp.int32),
        pltpu.VMEM((2, _G, HALF), jnp.float32),
        pltpu.VMEM((16,), jnp.int32),
        pltpu.VMEM((_ACC2,), jnp.float32),
        pltpu.VMEM((_ACC2,), jnp.float32),
        pltpu.VMEM((_ACC2,), jnp.float32),
        pltpu.VMEM((_ACC2,), jnp.float32),
        pltpu.SemaphoreType.DMA,
        pltpu.SemaphoreType.DMA,
        pltpu.SemaphoreType.DMA,
    ],
)(_sc_replay_body)


def _sc_replay(mc, msrc, mdl, B):
    Bstack = jnp.concatenate([B[:, :HALF], B[:, HALF:]], axis=0)
    S, S2, MN, MX = _sc_replay_call(mc, msrc, mdl, Bstack)

    def halves(a):
        a = a.reshape(2, _NPAD, HALF)
        return jnp.concatenate([a[0], a[1]], axis=-1)

    return halves(S), halves(S2), halves(MN), halves(MX)


def _combine_body(x_ref, A_ref, cnt_ref, SB_ref, S2B_ref, MN_ref, MX_ref,
                  Wp_ref, bp_ref, Wl_ref, bl_ref, o_ref):
    x = x_ref[...]
    A = A_ref[...]
    cnt = cnt_ref[...]
    SB = SB_ref[...]
    S2B = S2B_ref[...]
    MN = MN_ref[...]
    MX = MX_ref[...]

    denom = jnp.maximum(cnt, 1.0)
    inv = 1.0 / denom
    s = cnt * A + SB
    mean = s * inv
    mb = SB * inv
    varb = jnp.maximum(S2B * inv - mb * mb, 0.0)
    std = jnp.sqrt(varb + 1e-5)
    has = cnt > 0.0
    mn = jnp.where(has, A + MN, 0.0)
    mx = jnp.where(has, A + MX, 0.0)

    logd = jnp.log(denom + 1.0)
    amp = logd * (1.0 / AVG_DEG_LOG)
    att = AVG_DEG_LOG / logd

    hp = jnp.dot(x, Wp_ref[pl.ds(0, D), :], precision="default")
    pieces = (mean, s, std, mn, mx)
    for i, p in enumerate(pieces):
        off = D + i * D
        hp += jnp.dot(p, Wp_ref[pl.ds(off, D), :], precision="default")
    for i, p in enumerate(pieces):
        off = D + (5 + i) * D
        hp += jnp.dot(p * amp, Wp_ref[pl.ds(off, D), :], precision="default")
    for i, p in enumerate(pieces):
        off = D + (10 + i) * D
        hp += jnp.dot(p * att, Wp_ref[pl.ds(off, D), :], precision="default")
    hp += bp_ref[...]
    h = jnp.dot(hp, Wl_ref[...], precision="default") + bl_ref[...]
    o_ref[...] = jnp.maximum(h, 0.0)


def _combine(x, A, cnt, SB, S2B, MN, MX, W_post, b_post, W_lin, b_lin):
    grid = (_NPAD // _BLK,)
    row_spec = pl.BlockSpec((_BLK, D), lambda i: (i, 0))
    out = pl.pallas_call(
        _combine_body,
        grid=grid,
        in_specs=[
            row_spec,
            row_spec,
            pl.BlockSpec((_BLK, 1), lambda i: (i, 0)),
            row_spec, row_spec, row_spec, row_spec,
            pl.BlockSpec((16 * D, D), lambda i: (0, 0)),
            pl.BlockSpec((1, D), lambda i: (0, 0)),
            pl.BlockSpec((D, D), lambda i: (0, 0)),
            pl.BlockSpec((1, D), lambda i: (0, 0)),
        ],
        out_specs=row_spec,
        out_shape=jax.ShapeDtypeStruct((_NPAD, D), jnp.float32),
    )(x, A, cnt, SB, S2B, MN, MX, W_post, b_post[None, :], W_lin, b_lin[None, :])
    return out


def _pad_rows(a):
    return jnp.pad(a, ((0, _NPAD - N_NODES), (0, 0)))


def _layer(x, mc, msrc, mdl, cnt, W_pre, b_pre, W_post, b_post, W_lin, b_lin):
    A = jnp.dot(x, W_pre[:D], precision="default")
    B = jnp.dot(x, W_pre[D:], precision="default") + b_pre
    SB, S2B, MN, MX = _sc_replay(mc, msrc, mdl, B)
    h = _combine(_pad_rows(x), _pad_rows(A), cnt[:, None],
                 SB, S2B, MN, MX,
                 W_post, b_post, W_lin, b_lin)
    return h[:N_NODES]


def kernel(x, edge_index, W_pre1, b_pre1, W_post1, b_post1, W_lin1, b_lin1,
           W_pre2, b_pre2, W_post2, b_post2, W_lin2, b_lin2, W_out, b_out):
    src = edge_index[0].astype(jnp.int32)
    dst = edge_index[1].astype(jnp.int32)
    cnt2d, mc, msrc, mdl = _sc_build_call(dst, src)
    cnt = cnt2d.reshape(_NPAD)
    h = _layer(x, mc, msrc, mdl, cnt,
               W_pre1, b_pre1, W_post1, b_post1, W_lin1, b_lin1)
    h = _layer(h, mc, msrc, mdl, cnt,
               W_pre2, b_pre2, W_post2, b_post2, W_lin2, b_lin2)
    out = jnp.dot(h, W_out, precision="default") + b_out
    return jnp.squeeze(out, axis=-1)


# speedup vs baseline: 5.4276x; 1.0139x over previous
import functools
import math

import jax
import jax.numpy as jnp
from jax import lax
from jax.experimental import pallas as pl
from jax.experimental.pallas import tpu as pltpu
from jax.experimental.pallas import tpu_sc as plsc

D = 128
HALF = 64
N_NODES = 10000
N_EDGES = 320000
AVG_DEG_LOG = math.log(33.0)

_BLK = 256
_NPAD = 10240

_NT = 320
_CH = 3200
_NCH = N_EDGES // _CH
_CHV = _CH // 16
_G = 128
_MROWS = 32
_ACC = _NT * HALF
_ACC2 = (_NT + 1) * HALF

_FMAX = 3.4e38


def _sc_build_body(dst_hbm, src_hbm,
                   cnt_out, mc_out, msrc_out, mdl_out,
                   dstc, srcc, idxb, dlb, mcstg, cnta,
                   csem, wsem):
    c = lax.axis_index("c")
    s = lax.axis_index("s")
    r = c * 16 + s
    lo = r * _NT
    ones = jnp.ones((16,), jnp.float32)
    zf = jnp.zeros((16,), jnp.float32)

    def chunk_descs(ci):
        slot = ci & 1
        hsl = pl.ds(ci * _CH, _CH)
        return (pltpu.make_async_copy(dst_hbm.at[hsl], dstc.at[slot], csem),
                pltpu.make_async_copy(src_hbm.at[hsl], srcc.at[slot], csem))

    def chunk_start(ci):
        for dsc in chunk_descs(ci):
            dsc.start()

    def chunk_wait(ci):
        for dsc in chunk_descs(ci):
            dsc.wait()

    def write_descs(b):
        row = b & (_MROWS - 1)
        bsl = pl.ds(b * _G, _G)
        return (pltpu.make_async_copy(idxb.at[row], msrc_out.at[r, bsl],
                                      wsem),
                pltpu.make_async_copy(dlb.at[pl.ds(row * _G, _G)],
                                      mdl_out.at[r, bsl], wsem))

    def write_start(b):
        for dsc in write_descs(b):
            dsc.start()

    def write_wait(b):
        for dsc in write_descs(b):
            dsc.wait()

    def zero_cnt(i, _):
        cnta[pl.ds(i * 16, 16)] = zf
        return 0

    lax.fori_loop(0, _NT // 16, zero_cnt, 0)

    chunk_start(0)

    def wb_body(b, _):
        write_start(b)
        pl.when(b > 0)(lambda: write_wait(b - 1))
        return 0

    def chunk_body(ci, carry):
        mcnt, wdone = carry
        chunk_wait(ci)
        pl.when(ci + 1 < _NCH)(lambda: chunk_start(ci + 1))
        slot = ci & 1

        def vreg_body(v, mcnt):
            dstv = dstc[slot, pl.ds(v * 16, 16)]
            srcv = srcc[slot, pl.ds(v * 16, 16)]
            match = (dstv >= lo) & (dstv < lo + _NT)
            dloc = dstv - lo
            plsc.addupdate_scatter(cnta, [dloc], ones, mask=match)
            mi = jnp.where(match, 1, 0)
            ranks = plsc.cumsum(mi)
            pos = ranks + (mcnt - 1)
            plsc.store_scatter(idxb,
                               [(pos >> 7) & (_MROWS - 1), pos & 127],
                               srcv, mask=match)
            plsc.store_scatter(dlb, [pos & (_MROWS * _G - 1)],
                               dloc, mask=match)
            return mcnt + ranks[15]

        mcnt = lax.fori_loop(0, _CHV, vreg_body, mcnt, unroll=2)
        target = mcnt >> 7
        lax.fori_loop(wdone, target, wb_body, 0)
        return mcnt, jnp.maximum(wdone, target)

    mcnt, wdone = lax.fori_loop(0, _NCH, chunk_body, (0, 0))

    lanes = lax.iota(jnp.int32, 16)
    padto = ((mcnt + 15) >> 4) << 4
    ppos = mcnt + lanes
    pmask = ppos < padto
    plsc.store_scatter(idxb, [(ppos >> 7) & (_MROWS - 1), ppos & 127],
                       jnp.zeros((16,), jnp.int32), mask=pmask)
    plsc.store_scatter(dlb, [ppos & (_MROWS * _G - 1)],
                       jnp.full((16,), _NT, jnp.int32), mask=pmask)
    targf = (padto + 127) >> 7
    lax.fori_loop(wdone, targf, wb_body, 0)
    pl.when(targf > 0)(lambda: write_wait(targf - 1))

    mcstg[...] = jnp.zeros((16,), jnp.int32) + padto
    pltpu.sync_copy(mcstg, mc_out.at[r])
    pltpu.sync_copy(cnta, cnt_out.at[r])


_sc_build_call = functools.partial(
    pl.kernel,
    mesh=plsc.VectorSubcoreMesh(core_axis_name="c", subcore_axis_name="s"),
    compiler_params=pltpu.CompilerParams(needs_layout_passes=False,
                                         use_tc_tiling_on_sc=False),
    out_type=[
        jax.ShapeDtypeStruct((32, _NT), jnp.float32),
        jax.ShapeDtypeStruct((32, 16), jnp.int32),
        jax.ShapeDtypeStruct((32, N_EDGES), jnp.int32),
        jax.ShapeDtypeStruct((32, N_EDGES), jnp.int32),
    ],
    scratch_types=[
        pltpu.VMEM((2, _CH), jnp.int32),
        pltpu.VMEM((2, _CH), jnp.int32),
        pltpu.VMEM((_MROWS, _G), jnp.int32),
        pltpu.VMEM((_MROWS * _G,), jnp.int32),
        pltpu.VMEM((16,), jnp.int32),
        pltpu.VMEM((_NT,), jnp.float32),
        pltpu.SemaphoreType.DMA,
        pltpu.SemaphoreType.DMA,
    ],
)(_sc_build_body)


def _sc_replay_body(mc_hbm, msrc_hbm, mdl_hbm, B_hbm,
                    S_out, S2_out, MN_out, MX_out,
                    lsrc, ldl, gidx, rowb, mcb,
                    Sa, S2a, MNa, MXa,
                    lsem, gsem0, gsem1):
    c = lax.axis_index("c")
    s = lax.axis_index("s")
    zf = jnp.zeros((16,), jnp.float32)

    def pass_body(p, _):
        r = p * 16 + s

        def zero_body(i, _):
            off = pl.ds(i * 16, 16)
            Sa[off] = zf
            S2a[off] = zf
            MNa[off] = zf + _FMAX
            MXa[off] = zf - _FMAX
            return 0

        lax.fori_loop(0, _ACC2 // 16, zero_body, 0)

        pltpu.sync_copy(mc_hbm.at[r], mcb)
        padcnt = mcb[...][0]
        nb = (padcnt + 127) >> 7

        def list_descs(b):
            slot = b & 1
            bsl = pl.ds(b * _G, _G)
            return (pltpu.make_async_copy(msrc_hbm.at[r, bsl],
                                          lsrc.at[slot], lsem),
                    pltpu.make_async_copy(mdl_hbm.at[r, bsl],
                                          ldl.at[slot], lsem))

        def lstart(b):
            for dsc in list_descs(b):
                dsc.start()

        def lwait(b):
            for dsc in list_descs(b):
                dsc.wait()

        def prep_and_gather(b):
            slot = b & 1
            lwait(b)
            for t in range(8):
                tsl = pl.ds(t * 16, 16)
                gidx[slot, tsl] = lsrc[slot, tsl] + c * _NPAD

            def go(slot_sem, sl):
                pltpu.make_async_copy(
                    B_hbm.at[gidx.at[sl]], rowb.at[sl], slot_sem).start()

            pl.when((b & 1) == 0)(lambda: go(gsem0, 0))
            pl.when((b & 1) == 1)(lambda: go(gsem1, 1))

        def accumulate(b):
            slot = b & 1

            def wait(slot_sem, sl):
                pltpu.make_async_copy(
                    B_hbm.at[gidx.at[sl]], rowb.at[sl], slot_sem).wait()

            pl.when((b & 1) == 0)(lambda: wait(gsem0, 0))
            pl.when((b & 1) == 1)(lambda: wait(gsem1, 1))
            ng = jnp.minimum(padcnt - b * _G, _G) >> 4

            def group_body(g, _):
                dlv = ldl[slot, pl.ds(g * 16, 16)]
                for j in range(16):
                    base = dlv[j] * HALF
                    for k in range(4):
                        off = pl.ds(base + k * 16, 16)
                        rv = rowb[slot, g * 16 + j, pl.ds(k * 16, 16)]
                        plsc.addupdate(Sa.at[off], rv)
                        plsc.addupdate(S2a.at[off], rv * rv)
                        MNa[off] = jnp.minimum(MNa[off], rv)
                        MXa[off] = jnp.maximum(MXa[off], rv)
                return 0

            lax.fori_loop(0, ng, group_body, 0)

        def step(j, _):
            pl.when(j >= 2)(lambda: accumulate(j - 2))
            pl.when(j < nb)(lambda: lstart(j))
            pl.when((j >= 1) & (j <= nb))(lambda: prep_and_gather(j - 1))
            return 0

        lax.fori_loop(0, nb + 2, step, 0)

        asl = pl.ds(0, _ACC)
        pltpu.sync_copy(Sa.at[asl], S_out.at[c, r])
        pltpu.sync_copy(S2a.at[asl], S2_out.at[c, r])
        pltpu.sync_copy(MNa.at[asl], MN_out.at[c, r])
        pltpu.sync_copy(MXa.at[asl], MX_out.at[c, r])
        return 0

    lax.fori_loop(0, 2, pass_body, 0)


_sc_replay_call = functools.partial(
    pl.kernel,
    mesh=plsc.VectorSubcoreMesh(core_axis_name="c", subcore_axis_name="s"),
    compiler_params=pltpu.CompilerParams(needs_layout_passes=False,
                                         use_tc_tiling_on_sc=False),
    out_type=[
        jax.ShapeDtypeStruct((2, 32, _ACC), jnp.float32),
        jax.ShapeDtypeStruct((2, 32, _ACC), jnp.float32),
        jax.ShapeDtypeStruct((2, 32, _ACC), jnp.float32),
        jax.ShapeDtypeStruct((2, 32, _ACC), jnp.float32),
    ],
    scratch_types=[
        pltpu.VMEM((2, _G), jnp.int32),
        pltpu.VMEM((2, _G), jnp.int32),
        pltpu.VMEM((2, _G), jnp.int32),
        pltpu.VMEM((2, _G, HALF), jnp.float32),
        pltpu.VMEM((16,), jnp.int32),
        pltpu.VMEM((_ACC2,), jnp.float32),
        pltpu.VMEM((_ACC2,), jnp.float32),
        pltpu.VMEM((_ACC2,), jnp.float32),
        pltpu.VMEM((_ACC2,), jnp.float32),
        pltpu.SemaphoreType.DMA,
        pltpu.SemaphoreType.DMA,
        pltpu.SemaphoreType.DMA,
    ],
)(_sc_replay_body)


def _sc_replay(mc, msrc, mdl, Bst):
    S, S2, MN, MX = _sc_replay_call(mc, msrc, mdl,
                                    Bst.reshape(2 * _NPAD, HALF))
    return (S.reshape(2, _NPAD, HALF), S2.reshape(2, _NPAD, HALF),
            MN.reshape(2, _NPAD, HALF), MX.reshape(2, _NPAD, HALF))


def _pre_body(x_ref, Wpre_ref, bpre_ref, A_ref, Bst_ref):
    x = x_ref[...]
    A_ref[...] = jnp.dot(x, Wpre_ref[pl.ds(0, D), :], precision="default")
    B = jnp.dot(x, Wpre_ref[pl.ds(D, D), :], precision="default") + bpre_ref[...]
    Bst_ref[0, :, :] = B[:, :HALF]
    Bst_ref[1, :, :] = B[:, HALF:]


def _pre(x, W_pre, b_pre):
    grid = (_NPAD // _BLK,)
    return pl.pallas_call(
        _pre_body,
        grid=grid,
        in_specs=[
            pl.BlockSpec((_BLK, D), lambda i: (i, 0)),
            pl.BlockSpec((2 * D, D), lambda i: (0, 0)),
            pl.BlockSpec((1, D), lambda i: (0, 0)),
        ],
        out_specs=[
            pl.BlockSpec((_BLK, D), lambda i: (i, 0)),
            pl.BlockSpec((2, _BLK, HALF), lambda i: (0, i, 0)),
        ],
        out_shape=[
            jax.ShapeDtypeStruct((N_NODES, D), jnp.float32),
            jax.ShapeDtypeStruct((2, _NPAD, HALF), jnp.float32),
        ],
    )(x, W_pre, b_pre[None, :])


def _combine_body(x_ref, A_ref, cnt_ref, SB0_ref, SB1_ref, S2B0_ref,
                  S2B1_ref, MN0_ref, MN1_ref, MX0_ref, MX1_ref,
                  Wp_ref, bp_ref, Wl_ref, bl_ref, o_ref):
    x = x_ref[...]
    A = A_ref[...]
    cnt = cnt_ref[...]
    SB = jnp.concatenate([SB0_ref[...][0], SB1_ref[...][0]], axis=-1)
    S2B = jnp.concatenate([S2B0_ref[...][0], S2B1_ref[...][0]], axis=-1)
    MN = jnp.concatenate([MN0_ref[...][0], MN1_ref[...][0]], axis=-1)
    MX = jnp.concatenate([MX0_ref[...][0], MX1_ref[...][0]], axis=-1)

    denom = jnp.maximum(cnt, 1.0)
    inv = 1.0 / denom
    s = cnt * A + SB
    mean = s * inv
    mb = SB * inv
    varb = jnp.maximum(S2B * inv - mb * mb, 0.0)
    std = jnp.sqrt(varb + 1e-5)
    has = cnt > 0.0
    mn = jnp.where(has, A + MN, 0.0)
    mx = jnp.where(has, A + MX, 0.0)

    logd = jnp.log(denom + 1.0)
    amp = logd * (1.0 / AVG_DEG_LOG)
    att = AVG_DEG_LOG / logd

    hp = jnp.dot(x, Wp_ref[pl.ds(0, D), :], precision="default")
    pieces = (mean, s, std, mn, mx)
    for i, p in enumerate(pieces):
        off = D + i * D
        hp += jnp.dot(p, Wp_ref[pl.ds(off, D), :], precision="default")
    for i, p in enumerate(pieces):
        off = D + (5 + i) * D
        hp += jnp.dot(p * amp, Wp_ref[pl.ds(off, D), :], precision="default")
    for i, p in enumerate(pieces):
        off = D + (10 + i) * D
        hp += jnp.dot(p * att, Wp_ref[pl.ds(off, D), :], precision="default")
    hp += bp_ref[...]
    h = jnp.dot(hp, Wl_ref[...], precision="default") + bl_ref[...]
    o_ref[...] = jnp.maximum(h, 0.0)


def _combine(x, A, cnt, SB, S2B, MN, MX, W_post, b_post, W_lin, b_lin):
    grid = (_NPAD // _BLK,)
    row_spec = pl.BlockSpec((_BLK, D), lambda i: (i, 0))
    half0 = pl.BlockSpec((1, _BLK, HALF), lambda i: (0, i, 0))
    half1 = pl.BlockSpec((1, _BLK, HALF), lambda i: (1, i, 0))
    out = pl.pallas_call(
        _combine_body,
        grid=grid,
        in_specs=[
            row_spec,
            row_spec,
            pl.BlockSpec((_BLK, 1), lambda i: (i, 0)),
            half0, half1, half0, half1, half0, half1, half0, half1,
            pl.BlockSpec((16 * D, D), lambda i: (0, 0)),
            pl.BlockSpec((1, D), lambda i: (0, 0)),
            pl.BlockSpec((D, D), lambda i: (0, 0)),
            pl.BlockSpec((1, D), lambda i: (0, 0)),
        ],
        out_specs=row_spec,
        out_shape=jax.ShapeDtypeStruct((N_NODES, D), jnp.float32),
    )(x, A, cnt, SB, SB, S2B, S2B, MN, MN, MX, MX,
      W_post, b_post[None, :], W_lin, b_lin[None, :])
    return out


def _layer(x, mc, msrc, mdl, cnt, W_pre, b_pre, W_post, b_post, W_lin, b_lin):
    A, Bst = _pre(x, W_pre, b_pre)
    SB, S2B, MN, MX = _sc_replay(mc, msrc, mdl, Bst)
    return _combine(x, A, cnt[:, None], SB, S2B, MN, MX,
                    W_post, b_post, W_lin, b_lin)


def kernel(x, edge_index, W_pre1, b_pre1, W_post1, b_post1, W_lin1, b_lin1,
           W_pre2, b_pre2, W_post2, b_post2, W_lin2, b_lin2, W_out, b_out):
    src = edge_index[0].astype(jnp.int32)
    dst = edge_index[1].astype(jnp.int32)
    cnt2d, mc, msrc, mdl = _sc_build_call(dst, src)
    cnt = cnt2d.reshape(_NPAD)
    h = _layer(x, mc, msrc, mdl, cnt,
               W_pre1, b_pre1, W_post1, b_post1, W_lin1, b_lin1)
    h = _layer(h, mc, msrc, mdl, cnt,
               W_pre2, b_pre2, W_post2, b_post2, W_lin2, b_lin2)
    out = jnp.dot(h, W_out, precision="default") + b_out
    return jnp.squeeze(out, axis=-1)
